# baseline (speedup 1.0000x reference)
import functools

import jax
import jax.numpy as jnp
from jax import lax
from jax.experimental import pallas as pl
from jax.experimental.pallas import tpu as pltpu
from jax.experimental.pallas import tpu_sc as plsc

NC, NS, L = 2, 16, 16
NW = NC * NS
SEQ = 200
D = 64
B, T = 4096, 200
TOTAL = B * T
PER_W = TOTAL // NW
CHUNK = 400
IDX_W = 100
N_SUB = CHUNK // IDX_W
CHUNKS = PER_W // CHUNK


def _positional_encoding(nb_words, nb_dimensions):
    pos = jnp.arange(nb_words, dtype=jnp.float32)[:, None]
    dim = jnp.arange(nb_dimensions, dtype=jnp.float32)[None, :]
    ang = pos / jnp.power(10000.0, 2.0 * dim / nb_dimensions)
    temp1 = jnp.sin(ang)
    temp2 = jnp.cos(ang)
    Z = jnp.zeros((nb_words, nb_dimensions), dtype=jnp.float32)
    Z = Z.at[:, 0::2].set(temp1[:, 0::2])
    Z = Z.at[:, 1::2].set(temp2[:, 1::2])
    return Z


def _body(idx_hbm, table_hbm, pe_hbm, out_hbm, idx_v, rows_v, pe_v, gsem):
    wid = lax.axis_index("c") * NS + lax.axis_index("s")
    pltpu.sync_copy(pe_hbm, pe_v)

    def chunk_body(ci, _):
        irow0 = wid * (PER_W // IDX_W) + ci * N_SUB
        pltpu.sync_copy(idx_hbm.at[pl.ds(irow0, N_SUB)], idx_v)
        copies = []
        for j in range(N_SUB):
            copies.append(pltpu.async_copy(
                table_hbm.at[idx_v.at[j]],
                rows_v.at[pl.ds(j * IDX_W, IDX_W)], gsem))
        for c in copies:
            c.wait()

        def add_body(r, _):
            for cc in range(D // L):
                sl = pl.ds(cc * L, L)
                pv = pe_v[r, sl]
                for p in range(CHUNK // SEQ):
                    plsc.addupdate(rows_v.at[p * SEQ + r, sl], pv)
            return 0

        lax.fori_loop(0, SEQ, add_body, 0, unroll=False)
        pltpu.sync_copy(rows_v, out_hbm.at[pl.ds(wid * PER_W + ci * CHUNK, CHUNK)])
        return 0

    lax.fori_loop(0, CHUNKS, chunk_body, 0, unroll=False)


@jax.jit
def _embed(X2d, table, pe):
    k = pl.kernel(
        _body,
        out_type=jax.ShapeDtypeStruct((TOTAL, D), jnp.float32),
        mesh=plsc.VectorSubcoreMesh(core_axis_name="c", subcore_axis_name="s"),
        scratch_types=[
            pltpu.VMEM((N_SUB, IDX_W), jnp.int32),
            pltpu.VMEM((CHUNK, D), jnp.float32),
            pltpu.VMEM((SEQ, D), jnp.float32),
            pltpu.SemaphoreType.DMA,
        ],
        compiler_params=pltpu.CompilerParams(use_tc_tiling_on_sc=False),
    )
    return k(X2d, table, pe)


def kernel(X, table):
    pe = _positional_encoding(T, D)
    X2d = X.reshape(TOTAL // IDX_W, IDX_W)
    out = _embed(X2d, table, pe)
    return out.reshape(B, T, D)


# speedup vs baseline: 1.1074x; 1.1074x over previous
import jax
import jax.numpy as jnp
from jax import lax
from jax.experimental import pallas as pl
from jax.experimental.pallas import tpu as pltpu
from jax.experimental.pallas import tpu_sc as plsc

NC, NS, L = 2, 16, 16
NW = NC * NS
SEQ = 200
D = 64
B, T = 4096, 200
TOTAL = B * T
PER_W = TOTAL // NW
CHUNK = 800
CHUNKS = PER_W // CHUNK


def _positional_encoding(nb_words, nb_dimensions):
    pos = jnp.arange(nb_words, dtype=jnp.float32)[:, None]
    dim = jnp.arange(nb_dimensions, dtype=jnp.float32)[None, :]
    ang = pos / jnp.power(10000.0, 2.0 * dim / nb_dimensions)
    temp1 = jnp.sin(ang)
    temp2 = jnp.cos(ang)
    Z = jnp.zeros((nb_words, nb_dimensions), dtype=jnp.float32)
    Z = Z.at[:, 0::2].set(temp1[:, 0::2])
    Z = Z.at[:, 1::2].set(temp2[:, 1::2])
    return Z


def _body(idx_hbm, table_hbm, pe_hbm, out_hbm,
          idx0, idx1, rows0, rows1, pe_v,
          gsem0, gsem1, osem0, osem1):
    wid = lax.axis_index("c") * NS + lax.axis_index("s")
    base = wid * PER_W
    pltpu.sync_copy(pe_hbm, pe_v)

    bufs = [(idx0, rows0, gsem0, osem0), (idx1, rows1, gsem1, osem1)]

    def fire_gather(ci, b):
        idx_v, rows_v, gsem, _ = bufs[b]
        pltpu.sync_copy(idx_hbm.at[pl.ds(base + ci * CHUNK, CHUNK)], idx_v)
        return pltpu.async_copy(table_hbm.at[idx_v], rows_v, gsem)

    def add_pe(b):
        rows_v = bufs[b][1]

        def add_row(r, _):
            for cc in range(D // L):
                sl = pl.ds(cc * L, L)
                pv = pe_v[r, sl]
                for p in range(CHUNK // SEQ):
                    plsc.addupdate(rows_v.at[p * SEQ + r, sl], pv)
            return 0

        lax.fori_loop(0, SEQ, add_row, 0, unroll=False)

    def fire_out(ci, b):
        _, rows_v, _, osem = bufs[b]
        return pltpu.async_copy(
            rows_v, out_hbm.at[pl.ds(base + ci * CHUNK, CHUNK)], osem)

    def wait_gather(b):
        idx_v, rows_v, gsem, _ = bufs[b]
        pltpu.make_async_copy(table_hbm.at[idx_v], rows_v, gsem).wait()

    def wait_out(ci, b):
        _, rows_v, _, osem = bufs[b]
        pltpu.make_async_copy(
            rows_v, out_hbm.at[pl.ds(base + ci * CHUNK, CHUNK)], osem).wait()

    def step(ci, b, wait_prev_out):
        if wait_prev_out:
            wait_out(ci - 1, b ^ 1)
        fire_gather(ci + 1, b ^ 1)
        wait_gather(b)
        add_pe(b)
        fire_out(ci, b)

    fire_gather(0, 0)
    step(0, 0, wait_prev_out=False)
    step(1, 1, wait_prev_out=True)

    def pair(ci2, _):
        e = 2 * ci2
        step(e, 0, wait_prev_out=True)
        step(e + 1, 1, wait_prev_out=True)
        return 0

    lax.fori_loop(1, CHUNKS // 2 - 1, pair, 0, unroll=False)

    step(CHUNKS - 2, 0, wait_prev_out=True)
    wait_out(CHUNKS - 2, 0)
    wait_gather(1)
    add_pe(1)
    fire_out(CHUNKS - 1, 1)
    wait_out(CHUNKS - 1, 1)


@jax.jit
def _embed(Xf, table, pe):
    k = pl.kernel(
        _body,
        out_type=jax.ShapeDtypeStruct((TOTAL, D), jnp.float32),
        mesh=plsc.VectorSubcoreMesh(core_axis_name="c", subcore_axis_name="s"),
        scratch_types=[
            pltpu.VMEM((CHUNK,), jnp.int32),
            pltpu.VMEM((CHUNK,), jnp.int32),
            pltpu.VMEM((CHUNK, D), jnp.float32),
            pltpu.VMEM((CHUNK, D), jnp.float32),
            pltpu.VMEM((SEQ, D), jnp.float32),
            pltpu.SemaphoreType.DMA,
            pltpu.SemaphoreType.DMA,
            pltpu.SemaphoreType.DMA,
            pltpu.SemaphoreType.DMA,
        ],
        compiler_params=pltpu.CompilerParams(use_tc_tiling_on_sc=False),
    )
    return k(Xf, table, pe)


def kernel(X, table):
    pe = _positional_encoding(T, D)
    out = _embed(X.reshape(TOTAL), table, pe)
    return out.reshape(B, T, D)


# speedup vs baseline: 1.3701x; 1.2371x over previous
import jax
import jax.numpy as jnp
from jax import lax
from jax.experimental import pallas as pl
from jax.experimental.pallas import tpu as pltpu
from jax.experimental.pallas import tpu_sc as plsc

NC, NS, L = 2, 16, 16
NW = NC * NS
SEQ = 200
D = 64
B, T = 4096, 200
TOTAL = B * T
PER_W = TOTAL // NW
NSEQ = 2
CHUNK = NSEQ * SEQ
CHUNKS = PER_W // CHUNK
SEQ_PER_W = PER_W // SEQ


def _positional_encoding(nb_words, nb_dimensions):
    pos = jnp.arange(nb_words, dtype=jnp.float32)[:, None]
    dim = jnp.arange(nb_dimensions, dtype=jnp.float32)[None, :]
    ang = pos / jnp.power(10000.0, 2.0 * dim / nb_dimensions)
    temp1 = jnp.sin(ang)
    temp2 = jnp.cos(ang)
    Z = jnp.zeros((nb_words, nb_dimensions), dtype=jnp.float32)
    Z = Z.at[:, 0::2].set(temp1[:, 0::2])
    Z = Z.at[:, 1::2].set(temp2[:, 1::2])
    return Z


def _body(idx_hbm, table_hbm, pe_hbm, out_hbm,
          idx0, idx1, rows0, rows1, pe_v,
          gsem0, gsem1, osem0, osem1):
    wid = lax.axis_index("c") * NS + lax.axis_index("s")
    base = wid * PER_W
    sbase = wid * SEQ_PER_W
    pltpu.sync_copy(pe_hbm, pe_v)

    bufs = [(idx0, rows0, gsem0, osem0), (idx1, rows1, gsem1, osem1)]

    def fire_gather(ci, b):
        idx_v, rows_v, gsem, _ = bufs[b]
        pltpu.sync_copy(idx_hbm.at[pl.ds(base + ci * CHUNK, CHUNK)], idx_v)
        pltpu.async_copy(table_hbm.at[idx_v], rows_v, gsem)

    def wait_gather(b):
        idx_v, rows_v, gsem, _ = bufs[b]
        pltpu.make_async_copy(table_hbm.at[idx_v], rows_v, gsem).wait()

    def add_pe(b):
        rows_v = bufs[b][1]

        def row_body(r, _):
            for cc in range(D // L):
                sl = pl.ds(cc * L, L)
                pv = pe_v[r, sl]
                for s in range(NSEQ):
                    plsc.addupdate(rows_v.at[s * SEQ + r, sl], pv)
            return 0

        lax.fori_loop(0, SEQ, row_body, 0, unroll=False)

    def fire_out(ci, b):
        _, rows_v, _, osem = bufs[b]
        pltpu.async_copy(rows_v, out_hbm.at[pl.ds(base + ci * CHUNK, CHUNK)],
                         osem)

    def wait_out(ci, b):
        _, rows_v, _, osem = bufs[b]
        pltpu.make_async_copy(rows_v,
                              out_hbm.at[pl.ds(base + ci * CHUNK, CHUNK)],
                              osem).wait()

    def step(ci, b, wait_prev_out):
        if wait_prev_out:
            wait_out(ci - 1, b ^ 1)
        fire_gather(ci + 1, b ^ 1)
        wait_gather(b)
        add_pe(b)
        fire_out(ci, b)

    fire_gather(0, 0)
    step(0, 0, wait_prev_out=False)
    step(1, 1, wait_prev_out=True)

    def pair(ci2, _):
        e = 2 * ci2
        step(e, 0, wait_prev_out=True)
        step(e + 1, 1, wait_prev_out=True)
        return 0

    lax.fori_loop(1, CHUNKS // 2 - 1, pair, 0, unroll=False)

    step(CHUNKS - 2, 0, wait_prev_out=True)
    wait_out(CHUNKS - 2, 0)
    wait_gather(1)
    add_pe(1)
    fire_out(CHUNKS - 1, 1)
    wait_out(CHUNKS - 1, 1)


@jax.jit
def _embed(X, table):
    pe = _positional_encoding(T, D)
    Xf = X.reshape(TOTAL)
    table_wide = jnp.pad(table, ((0, 0), (0, 2 * D - D)))
    k = pl.kernel(
        _body,
        out_type=jax.ShapeDtypeStruct((TOTAL, 2 * D), jnp.float32),
        mesh=plsc.VectorSubcoreMesh(core_axis_name="c", subcore_axis_name="s"),
        scratch_types=[
            pltpu.VMEM((CHUNK,), jnp.int32),
            pltpu.VMEM((CHUNK,), jnp.int32),
            pltpu.VMEM((CHUNK, 2 * D), jnp.float32),
            pltpu.VMEM((CHUNK, 2 * D), jnp.float32),
            pltpu.VMEM((SEQ, D), jnp.float32),
            pltpu.SemaphoreType.DMA,
            pltpu.SemaphoreType.DMA,
            pltpu.SemaphoreType.DMA,
            pltpu.SemaphoreType.DMA,
        ],
        compiler_params=pltpu.CompilerParams(use_tc_tiling_on_sc=True),
    )
    return k(Xf, table_wide, pe)


def kernel(X, table):
    out_wide = _embed(X, table)
    return out_wide[:, :D].reshape(B, T, D)
